# baseline (speedup 1.0000x reference)
import functools

import jax
import jax.numpy as jnp
from jax import lax
from jax.experimental import pallas as pl
from jax.experimental.pallas import tpu as pltpu
from jax.experimental.pallas import tpu_sc as plsc

NC = 2
NS = 16
NW = NC * NS
L = 16

D = 128
CHUNK = 64
NCHUNK0 = 312
NCHUNK1 = 8
NCT = NS * (NCHUNK0 + NCHUNK1)
NPAD = 10240
RPT = NPAD // NS

NROW = 4
NIB = 8


def _sc_agg_body(x_hbm, src_hbm, tgt_hbm, p_hbm, deg_hbm,
                 src_v, tgt_v, rows_v, deg_v, accum_sh, *sems):
    row_sems = sems[:NROW]
    sc_sems = sems[NROW:2 * NROW]
    idx_sems = sems[2 * NROW:]
    cid = lax.axis_index("c")
    tid = lax.axis_index("s")
    wid = tid * NC + cid
    zero16 = jnp.zeros((L,), jnp.float32)
    ones16 = jnp.ones((L,), jnp.float32)
    base = jnp.where(cid == 0, tid * NCHUNK0, NS * NCHUNK0 + tid * NCHUNK1)
    ncs = jnp.where(cid == 0, NCHUNK0, NCHUNK1)

    def _zrow(i, carry):
        for k in range(D // L):
            rows_v[0, i, pl.ds(k * L, L)] = zero16
        return carry
    lax.fori_loop(0, CHUNK, _zrow, 0)
    for i in range(RPT // CHUNK):
        pltpu.sync_copy(rows_v.at[0],
                        accum_sh.at[pl.ds(tid * RPT + i * CHUNK, CHUNK)])

    def _zdeg(i, carry):
        deg_v[pl.ds(i * L, L)] = zero16
        return carry
    lax.fori_loop(0, NPAD // L, _zdeg, 0)

    plsc.subcore_barrier()

    def _fire_idx(j, s):
        pltpu.async_copy(src_hbm.at[base + j], src_v.at[s], idx_sems[s])
        pltpu.async_copy(tgt_hbm.at[base + j], tgt_v.at[s], idx_sems[s])

    def _wait_idx(j, s):
        pltpu.make_async_copy(src_hbm.at[base + j], src_v.at[s],
                              idx_sems[s]).wait()
        pltpu.make_async_copy(tgt_hbm.at[base + j], tgt_v.at[s],
                              idx_sems[s]).wait()

    def _fire_rows(j, s, b):
        pltpu.async_copy(x_hbm.at[src_v.at[s]], rows_v.at[b], row_sems[b])

    def _wait_rows(s, b):
        pltpu.make_async_copy(x_hbm.at[src_v.at[s]], rows_v.at[b],
                              row_sems[b]).wait()

    def _wait_scatter(s, b):
        pltpu.make_async_copy(rows_v.at[b], accum_sh.at[tgt_v.at[s]],
                              sc_sems[b]).wait()

    for j in range(NIB):
        _fire_idx(j, j)
    for b in range(2):
        _wait_idx(b, b)
        _fire_rows(b, b, b)

    def _drain(j, s, b):
        _wait_rows(s, b)
        @pl.when(j < ncs - 2)
        def _():
            pltpu.async_copy(rows_v.at[b], accum_sh.at[tgt_v.at[s]],
                             sc_sems[b], add=True)
        @pl.when(j >= ncs - 2)
        def _():
            pltpu.sync_copy(rows_v.at[b], accum_sh.at[tgt_v.at[s]], add=True)
        for k in range(CHUNK // L):
            idx = tgt_v[s, pl.ds(k * L, L)]
            plsc.addupdate_scatter(deg_v, [idx], ones16)
        @pl.when(j >= 2)
        def _():
            _wait_scatter((s + 6) % NIB, (s + 2) % NROW)
        @pl.when((j >= 2) & (j + 6 < ncs))
        def _():
            _fire_idx(j + 6, (s + 6) % NIB)
        @pl.when(j + 2 < ncs)
        def _():
            _wait_idx(j + 2, (s + 2) % NIB)
            _fire_rows(j + 2, (s + 2) % NIB, (s + 2) % NROW)

    def _steady(i, carry):
        for jj in range(NIB):
            _drain(i * NIB + jj, jj, jj % NROW)
        return carry
    lax.fori_loop(0, ncs // NIB, _steady, 0)

    plsc.subcore_barrier()

    for i in range(RPT // CHUNK):
        off = tid * RPT + i * CHUNK
        pltpu.sync_copy(accum_sh.at[pl.ds(off, CHUNK)], rows_v.at[0])
        pltpu.sync_copy(rows_v.at[0], p_hbm.at[cid, pl.ds(off, CHUNK)])
    pltpu.sync_copy(deg_v, deg_hbm.at[wid])


_sc_agg = pl.kernel(
    _sc_agg_body,
    out_type=[
        jax.ShapeDtypeStruct((NC, NPAD, D), jnp.float32),
        jax.ShapeDtypeStruct((NW, NPAD), jnp.float32),
    ],
    mesh=plsc.VectorSubcoreMesh(core_axis_name="c", subcore_axis_name="s",
                                num_cores=NC, num_subcores=NS),
    compiler_params=pltpu.CompilerParams(needs_layout_passes=False),
    scratch_types=[
        pltpu.VMEM((NIB, CHUNK), jnp.int32),
        pltpu.VMEM((NIB, CHUNK), jnp.int32),
        pltpu.VMEM((NROW, CHUNK, D), jnp.float32),
        pltpu.VMEM((NPAD,), jnp.float32),
        pltpu.VMEM_SHARED((NPAD, D), jnp.float32),
    ] + [pltpu.SemaphoreType.DMA] * (2 * NROW + NIB),
)


def _combine_body(p_ref, deg_ref, w_ref, b_ref, o_ref):
    agg = p_ref[0] + p_ref[1]
    deg = jnp.sum(deg_ref[...], axis=0)
    y = lax.dot_general(agg, w_ref[...], (((1,), (1,)), ((), ())),
                        preferred_element_type=jnp.float32,
                        precision=lax.Precision.HIGHEST)
    y = y + deg[:, None] * b_ref[...]
    o_ref[...] = y / jnp.maximum(deg + 1.0, 1.0)[:, None]


def _combine(p, degs, W, b2d):
    R = 1280
    grid = NPAD // R
    return pl.pallas_call(
        _combine_body,
        grid=(grid,),
        in_specs=[
            pl.BlockSpec((NC, R, D), lambda i: (0, i, 0)),
            pl.BlockSpec((NW, R), lambda i: (0, i)),
            pl.BlockSpec((D, D), lambda i: (0, 0)),
            pl.BlockSpec((1, D), lambda i: (0, 0)),
        ],
        out_specs=pl.BlockSpec((R, D), lambda i: (i, 0)),
        out_shape=jax.ShapeDtypeStruct((NPAD, D), jnp.float32),
    )(p, degs, W, b2d)


@jax.jit
def kernel(x, edge_index, W, b):
    n = x.shape[0]
    e = edge_index.shape[1]
    src = edge_index[0].astype(jnp.int32)
    tgt = edge_index[1].astype(jnp.int32)
    e_pad = NCT * CHUNK
    src_p = jnp.concatenate(
        [src, jnp.zeros((e_pad - e,), jnp.int32)]).reshape(NCT, CHUNK)
    tgt_p = jnp.concatenate(
        [tgt, jnp.full((e_pad - e,), n, jnp.int32)]).reshape(NCT, CHUNK)
    p, degs = _sc_agg(x, src_p, tgt_p)
    out = _combine(p, degs, W, b.reshape(1, D))
    return out[:n]


# speedup vs baseline: 1.0924x; 1.0924x over previous
import functools

import jax
import jax.numpy as jnp
from jax import lax
from jax.experimental import pallas as pl
from jax.experimental.pallas import tpu as pltpu
from jax.experimental.pallas import tpu_sc as plsc

NC = 2
NS = 16
NW = NC * NS
L = 16

D = 128
CHUNK = 64
NCHUNK0 = 304
NCHUNK1 = 16
NCT = NS * (NCHUNK0 + NCHUNK1)
NPAD = 10240
RPT = NPAD // NS

NROW = 4
NIB = 8


def _sc_agg_body(x_hbm, src_hbm, tgt_hbm, p_hbm, deg_hbm,
                 src_v, tgt_v, rows_v, deg_v, accum_sh, *sems):
    row_sems = sems[:NROW]
    sc_sems = sems[NROW:2 * NROW]
    idx_sems = sems[2 * NROW:]
    cid = lax.axis_index("c")
    tid = lax.axis_index("s")
    wid = tid * NC + cid
    zero16 = jnp.zeros((L,), jnp.float32)
    ones16 = jnp.ones((L,), jnp.float32)
    base = jnp.where(cid == 0, tid * NCHUNK0, NS * NCHUNK0 + tid * NCHUNK1)
    ncs = jnp.where(cid == 0, NCHUNK0, NCHUNK1)

    def _zrow(i, carry):
        for k in range(D // L):
            rows_v[0, i, pl.ds(k * L, L)] = zero16
        return carry
    lax.fori_loop(0, CHUNK, _zrow, 0)
    for i in range(RPT // CHUNK):
        pltpu.sync_copy(rows_v.at[0],
                        accum_sh.at[pl.ds(tid * RPT + i * CHUNK, CHUNK)])

    def _zdeg(i, carry):
        deg_v[pl.ds(i * L, L)] = zero16
        return carry
    lax.fori_loop(0, NPAD // L, _zdeg, 0)

    plsc.subcore_barrier()

    def _fire_idx(j, s):
        pltpu.async_copy(src_hbm.at[base + j], src_v.at[s], idx_sems[s])
        pltpu.async_copy(tgt_hbm.at[base + j], tgt_v.at[s], idx_sems[s])

    def _wait_idx(j, s):
        pltpu.make_async_copy(src_hbm.at[base + j], src_v.at[s],
                              idx_sems[s]).wait()
        pltpu.make_async_copy(tgt_hbm.at[base + j], tgt_v.at[s],
                              idx_sems[s]).wait()

    def _fire_rows(j, s, b):
        pltpu.async_copy(x_hbm.at[src_v.at[s]], rows_v.at[b], row_sems[b])

    def _wait_rows(s, b):
        pltpu.make_async_copy(x_hbm.at[src_v.at[s]], rows_v.at[b],
                              row_sems[b]).wait()

    def _wait_scatter(s, b):
        pltpu.make_async_copy(rows_v.at[b], accum_sh.at[tgt_v.at[s]],
                              sc_sems[b]).wait()

    for j in range(NIB):
        _fire_idx(j, j)
    for b in range(2):
        _wait_idx(b, b)
        _fire_rows(b, b, b)

    def _drain(j, s, b):
        _wait_rows(s, b)
        @pl.when(j < ncs - 2)
        def _():
            pltpu.async_copy(rows_v.at[b], accum_sh.at[tgt_v.at[s]],
                             sc_sems[b], add=True)
        @pl.when(j >= ncs - 2)
        def _():
            pltpu.sync_copy(rows_v.at[b], accum_sh.at[tgt_v.at[s]], add=True)
        for k in range(CHUNK // L):
            idx = tgt_v[s, pl.ds(k * L, L)]
            plsc.addupdate_scatter(deg_v, [idx], ones16)
        @pl.when(j >= 2)
        def _():
            _wait_scatter((s + 6) % NIB, (s + 2) % NROW)
        @pl.when((j >= 2) & (j + 6 < ncs))
        def _():
            _fire_idx(j + 6, (s + 6) % NIB)
        @pl.when(j + 2 < ncs)
        def _():
            _wait_idx(j + 2, (s + 2) % NIB)
            _fire_rows(j + 2, (s + 2) % NIB, (s + 2) % NROW)

    def _steady(i, carry):
        for jj in range(NIB):
            _drain(i * NIB + jj, jj, jj % NROW)
        return carry
    lax.fori_loop(0, ncs // NIB, _steady, 0)

    plsc.subcore_barrier()

    for i in range(RPT // CHUNK):
        off = tid * RPT + i * CHUNK
        pltpu.sync_copy(accum_sh.at[pl.ds(off, CHUNK)], rows_v.at[0])
        pltpu.sync_copy(rows_v.at[0], p_hbm.at[cid, pl.ds(off, CHUNK)])
    pltpu.sync_copy(deg_v, deg_hbm.at[wid])


_sc_agg = pl.kernel(
    _sc_agg_body,
    out_type=[
        jax.ShapeDtypeStruct((NC, NPAD, D), jnp.float32),
        jax.ShapeDtypeStruct((NW, NPAD), jnp.float32),
    ],
    mesh=plsc.VectorSubcoreMesh(core_axis_name="c", subcore_axis_name="s",
                                num_cores=NC, num_subcores=NS),
    compiler_params=pltpu.CompilerParams(needs_layout_passes=False),
    scratch_types=[
        pltpu.VMEM((NIB, CHUNK), jnp.int32),
        pltpu.VMEM((NIB, CHUNK), jnp.int32),
        pltpu.VMEM((NROW, CHUNK, D), jnp.float32),
        pltpu.VMEM((NPAD,), jnp.float32),
        pltpu.VMEM_SHARED((NPAD, D), jnp.float32),
    ] + [pltpu.SemaphoreType.DMA] * (2 * NROW + NIB),
)


def _combine_body(p_ref, deg_ref, w_ref, b_ref, o_ref):
    agg = p_ref[0] + p_ref[1]
    deg = jnp.sum(deg_ref[...], axis=0)
    y = lax.dot_general(agg, w_ref[...], (((1,), (1,)), ((), ())),
                        preferred_element_type=jnp.float32,
                        precision=lax.Precision.HIGHEST)
    y = y + deg[:, None] * b_ref[...]
    o_ref[...] = y / jnp.maximum(deg + 1.0, 1.0)[:, None]


def _combine(p, degs, W, b2d):
    R = 1280
    grid = NPAD // R
    return pl.pallas_call(
        _combine_body,
        grid=(grid,),
        in_specs=[
            pl.BlockSpec((NC, R, D), lambda i: (0, i, 0)),
            pl.BlockSpec((NW, R), lambda i: (0, i)),
            pl.BlockSpec((D, D), lambda i: (0, 0)),
            pl.BlockSpec((1, D), lambda i: (0, 0)),
        ],
        out_specs=pl.BlockSpec((R, D), lambda i: (i, 0)),
        out_shape=jax.ShapeDtypeStruct((NPAD, D), jnp.float32),
    )(p, degs, W, b2d)


@jax.jit
def kernel(x, edge_index, W, b):
    n = x.shape[0]
    e = edge_index.shape[1]
    src = edge_index[0].astype(jnp.int32)
    tgt = edge_index[1].astype(jnp.int32)
    e_pad = NCT * CHUNK
    src_p = jnp.concatenate(
        [src, jnp.zeros((e_pad - e,), jnp.int32)]).reshape(NCT, CHUNK)
    tgt_p = jnp.concatenate(
        [tgt, jnp.full((e_pad - e,), n, jnp.int32)]).reshape(NCT, CHUNK)
    p, degs = _sc_agg(x, src_p, tgt_p)
    out = _combine(p, degs, W, b.reshape(1, D))
    return out[:n]
